# Initial kernel scaffold; baseline (speedup 1.0000x reference)
import functools

import jax
import jax.numpy as jnp
from jax import lax
from jax.experimental import pallas as pl
from jax.experimental.pallas import tpu as pltpu
from jax.experimental.pallas import tpu_sc as plsc

N = 10000
E = 320000
D = 128
NC = 2
NS = 16
NW = NC * NS
EPT = E // NW
CH = 80
NCHUNK = EPT // CH
RPT = N // NS
ZR = 125


def _sc_aggregate(x, src, dst):
    mesh = plsc.VectorSubcoreMesh(core_axis_name="c", subcore_axis_name="s")

    @functools.partial(
        pl.kernel,
        out_type=(
            jax.ShapeDtypeStruct((NC * N, D), jnp.float32),
            jax.ShapeDtypeStruct((NC * N, 16), jnp.float32),
        ),
        mesh=mesh,
        scratch_types=[
            pltpu.VMEM((CH,), jnp.int32),
            pltpu.VMEM((CH,), jnp.int32),
            pltpu.VMEM((CH, D), jnp.float32),
            pltpu.VMEM((CH, 16), jnp.float32),
            pltpu.VMEM((ZR, D), jnp.float32),
            pltpu.VMEM((ZR, 16), jnp.float32),
            pltpu.VMEM_SHARED((N, D), jnp.float32),
            pltpu.VMEM_SHARED((N, 16), jnp.float32),
            pltpu.SemaphoreType.DMA,
        ],
    )
    def k(x_hbm, src_hbm, dst_hbm, agg_out, deg_out,
          sidx, didx, rows, ones, zrow, zdeg, agg_sh, deg_sh, sem):
        cid = lax.axis_index("c")
        sid = lax.axis_index("s")
        wid = sid * NC + cid

        zero16 = jnp.zeros((16,), jnp.float32)
        one16 = jnp.ones((16,), jnp.float32)

        @pl.loop(0, ZR)
        def _(r):
            @pl.loop(0, D // 16)
            def _(c16):
                zrow[r, pl.ds(c16 * 16, 16)] = zero16
            zdeg[r, :] = zero16

        @pl.loop(0, CH)
        def _(r):
            ones[r, :] = one16

        @pl.loop(0, RPT // ZR)
        def _(t):
            off = sid * RPT + t * ZR
            pltpu.sync_copy(zrow, agg_sh.at[pl.ds(off, ZR)])
            pltpu.sync_copy(zdeg, deg_sh.at[pl.ds(off, ZR)])
        plsc.subcore_barrier()

        @pl.loop(0, NCHUNK)
        def _(cnum):
            base = wid * EPT + cnum * CH
            pltpu.sync_copy(src_hbm.at[pl.ds(base, CH)], sidx)
            pltpu.sync_copy(dst_hbm.at[pl.ds(base, CH)], didx)
            pltpu.async_copy(x_hbm.at[sidx], rows, sem).wait()
            pltpu.sync_copy(rows, agg_sh.at[didx], add=True)
            pltpu.sync_copy(ones, deg_sh.at[didx], add=True)

        plsc.subcore_barrier()
        off = sid * RPT
        pltpu.sync_copy(agg_sh.at[pl.ds(off, RPT)],
                        agg_out.at[pl.ds(cid * N + off, RPT)])
        pltpu.sync_copy(deg_sh.at[pl.ds(off, RPT)],
                        deg_out.at[pl.ds(cid * N + off, RPT)])

    return k(x, src, dst)


def _tc_dense(x, aggp, degp, W, b, gamma, beta):
    BR = 1000
    G = N // BR

    def body(x_ref, a0_ref, a1_ref, d0_ref, d1_ref, w_ref, b_ref, g_ref,
             be_ref, o_ref):
        deg = jnp.maximum(d0_ref[:, 0:1] + d1_ref[:, 0:1], 1.0)
        agg = (a0_ref[...] + a1_ref[...]) / deg
        w = w_ref[...]
        h = lax.dot_general(x_ref[...], w[:, :D], (((1,), (1,)), ((), ())),
                            preferred_element_type=jnp.float32)
        h = h + lax.dot_general(agg, w[:, D:], (((1,), (1,)), ((), ())),
                                preferred_element_type=jnp.float32)
        h = jnp.maximum(h + b_ref[...], 0.0)
        mu = jnp.mean(h, axis=-1, keepdims=True)
        hc = h - mu
        var = jnp.mean(hc * hc, axis=-1, keepdims=True)
        out = hc * lax.rsqrt(var + 1e-5)
        o_ref[...] = out * g_ref[...] + be_ref[...]

    return pl.pallas_call(
        body,
        grid=(G,),
        in_specs=[
            pl.BlockSpec((BR, D), lambda i: (i, 0)),
            pl.BlockSpec((BR, D), lambda i: (i, 0)),
            pl.BlockSpec((BR, D), lambda i: (i + G, 0)),
            pl.BlockSpec((BR, 16), lambda i: (i, 0)),
            pl.BlockSpec((BR, 16), lambda i: (i + G, 0)),
            pl.BlockSpec((D, 2 * D), lambda i: (0, 0)),
            pl.BlockSpec((1, D), lambda i: (0, 0)),
            pl.BlockSpec((1, D), lambda i: (0, 0)),
            pl.BlockSpec((1, D), lambda i: (0, 0)),
        ],
        out_specs=pl.BlockSpec((BR, D), lambda i: (i, 0)),
        out_shape=jax.ShapeDtypeStruct((N, D), jnp.float32),
    )(x, aggp, aggp, degp, degp, W, b.reshape(1, D), gamma.reshape(1, D),
      beta.reshape(1, D))


def kernel(x, edge_index, W, b, gamma, beta):
    ei = edge_index.astype(jnp.int32)
    aggp, degp = _sc_aggregate(x, ei[1], ei[0])
    return _tc_dense(x, aggp, degp, W, b, gamma, beta)


# speedup vs baseline: 4.9626x; 4.9626x over previous
import dataclasses
import functools

import jax
import jax.numpy as jnp
from jax import lax
from jax.experimental import pallas as pl
from jax.experimental.pallas import tpu as pltpu
from jax.experimental.pallas import tpu_sc as plsc

N = 10000
E = 320000
D = 128
NC = 2
NS = 16
NW = NC * NS
EPT = E // NW
CH = 80
NCHUNK = EPT // CH
STR = 640
NPAD = NS * STR
DM = 2000
DROWS = 2048
DSTR = DROWS // NS
REG = 14000
AOFF = 11000
OUT_ROWS = NC * REG


def _sc_aggregate(x, src, dst):
    mesh = plsc.VectorSubcoreMesh(core_axis_name="c", subcore_axis_name="s")
    cp = pltpu.CompilerParams()
    if "needs_layout_passes" in pltpu.CompilerParams.__dataclass_fields__:
        cp = dataclasses.replace(cp, needs_layout_passes=False)

    @functools.partial(
        pl.kernel,
        out_type=jax.ShapeDtypeStruct((OUT_ROWS, D), jnp.float32),
        mesh=mesh,
        compiler_params=cp,
        scratch_types=[
            pltpu.VMEM((CH,), jnp.int32),
            pltpu.VMEM((CH,), jnp.int32),
            pltpu.VMEM((CH,), jnp.int32),
            pltpu.VMEM((CH, D), jnp.float32),
            pltpu.VMEM((CH, D), jnp.float32),
            pltpu.VMEM_SHARED((NPAD, D), jnp.float32),
            pltpu.VMEM_SHARED((DROWS, D), jnp.float32),
            pltpu.SemaphoreType.DMA,
        ],
    )
    def k(x_hbm, src_hbm, dst_hbm, out_hbm,
          sidx, didx, dmod, rows, hot, agg_sh, deg_sh, sem):
        cid = lax.axis_index("c")
        sid = lax.axis_index("s")
        wid = sid * NC + cid
        reg = cid * REG

        zero16 = jnp.zeros((16,), jnp.float32)
        one16 = jnp.ones((16,), jnp.float32)
        lane = lax.iota(jnp.int32, 16)

        @pl.loop(0, CH)
        def _(r):
            @pl.loop(0, D // 16)
            def _(c16):
                rows[r, pl.ds(c16 * 16, 16)] = zero16
                hot[r, pl.ds(c16 * 16, 16)] = zero16

        for t in range(STR // CH):
            pltpu.sync_copy(rows, agg_sh.at[pl.ds(sid * STR + t * CH, CH)])
        for t in range(DSTR // CH + 1):
            nrow = CH if (t + 1) * CH <= DSTR else DSTR - t * CH
            pltpu.sync_copy(rows.at[pl.ds(0, nrow)],
                            deg_sh.at[pl.ds(sid * DSTR + t * CH, nrow)])
        plsc.subcore_barrier()

        @pl.loop(0, NCHUNK)
        def _(cnum):
            base = wid * EPT + cnum * CH
            pltpu.sync_copy(src_hbm.at[pl.ds(base, CH)], sidx)
            pltpu.sync_copy(dst_hbm.at[pl.ds(base, CH)], didx)
            pltpu.async_copy(x_hbm.at[sidx], rows, sem).wait()
            pltpu.sync_copy(rows, agg_sh.at[didx], add=True)

            for g in range(CH // 16):
                dv = didx[pl.ds(g * 16, 16)]
                dmod[pl.ds(g * 16, 16)] = lax.rem(dv, DM)
                col = (lax.div(dv, DM) * 16).astype(jnp.int32)
                row = lane + g * 16
                plsc.store_scatter(hot, [row, col], one16)
            pltpu.sync_copy(hot, deg_sh.at[dmod], add=True)
            for g in range(CH // 16):
                dv = didx[pl.ds(g * 16, 16)]
                col = (lax.div(dv, DM) * 16).astype(jnp.int32)
                row = lane + g * 16
                plsc.store_scatter(hot, [row, col], zero16)

        plsc.subcore_barrier()

        pltpu.sync_copy(agg_sh.at[pl.ds(sid * STR, STR)],
                        out_hbm.at[pl.ds(reg + sid * STR, STR)])
        pltpu.sync_copy(deg_sh.at[pl.ds(sid * DSTR, DSTR)],
                        out_hbm.at[pl.ds(reg + AOFF + sid * DSTR, DSTR)])

    return k(x, src, dst)


def _tc_dense(x, parts, W, b, gamma, beta):
    BR = 1000
    G = N // BR
    RB = REG // BR
    AB = AOFF // BR

    def body(x_ref, a0_ref, a1_ref, d0_ref, d1_ref, w_ref, b_ref, g_ref,
             be_ref, o_ref):
        i = pl.program_id(0)
        grp = i // 2
        lane_grp = jax.lax.broadcasted_iota(jnp.int32, (BR, D), 1) // 16
        mask = (lane_grp == grp).astype(jnp.float32)
        deg = jnp.sum((d0_ref[...] + d1_ref[...]) * mask, axis=-1,
                      keepdims=True)
        deg = jnp.maximum(deg, 1.0)
        agg = (a0_ref[...] + a1_ref[...]) / deg
        w = w_ref[...]
        h = lax.dot_general(x_ref[...], w[:, :D], (((1,), (1,)), ((), ())),
                            preferred_element_type=jnp.float32)
        h = h + lax.dot_general(agg, w[:, D:], (((1,), (1,)), ((), ())),
                                preferred_element_type=jnp.float32)
        h = jnp.maximum(h + b_ref[...], 0.0)
        mu = jnp.mean(h, axis=-1, keepdims=True)
        hc = h - mu
        var = jnp.mean(hc * hc, axis=-1, keepdims=True)
        out = hc * lax.rsqrt(var + 1e-5)
        o_ref[...] = out * g_ref[...] + be_ref[...]

    return pl.pallas_call(
        body,
        grid=(G,),
        in_specs=[
            pl.BlockSpec((BR, D), lambda i: (i, 0)),
            pl.BlockSpec((BR, D), lambda i: (i, 0)),
            pl.BlockSpec((BR, D), lambda i: (i + RB, 0)),
            pl.BlockSpec((BR, D), lambda i: (AB + i % 2, 0)),
            pl.BlockSpec((BR, D), lambda i: (AB + RB + i % 2, 0)),
            pl.BlockSpec((D, 2 * D), lambda i: (0, 0)),
            pl.BlockSpec((1, D), lambda i: (0, 0)),
            pl.BlockSpec((1, D), lambda i: (0, 0)),
            pl.BlockSpec((1, D), lambda i: (0, 0)),
        ],
        out_specs=pl.BlockSpec((BR, D), lambda i: (i, 0)),
        out_shape=jax.ShapeDtypeStruct((N, D), jnp.float32),
    )(x, parts, parts, parts, parts, W, b.reshape(1, D), gamma.reshape(1, D),
      beta.reshape(1, D))


def kernel(x, edge_index, W, b, gamma, beta):
    ei = edge_index.astype(jnp.int32)
    parts = _sc_aggregate(x, ei[1], ei[0])
    return _tc_dense(x, parts, W, b, gamma, beta)


# speedup vs baseline: 8.0168x; 1.6154x over previous
import dataclasses
import functools

import jax
import jax.numpy as jnp
from jax import lax
from jax.experimental import pallas as pl
from jax.experimental.pallas import tpu as pltpu
from jax.experimental.pallas import tpu_sc as plsc

N = 10000
E = 320000
D = 128
NC = 2
NS = 16
NW = NC * NS
EPT = E // NW
CH = 80
NCHUNK = EPT // CH
STR = 640
NPAD = NS * STR
DM = 2000
DROWS = 2048
DSTR = DROWS // NS
REG = 14000
AOFF = 11000
OUT_ROWS = NC * REG


def _sc_aggregate(x, src, dst):
    mesh = plsc.VectorSubcoreMesh(core_axis_name="c", subcore_axis_name="s")
    cp = pltpu.CompilerParams()
    if "needs_layout_passes" in pltpu.CompilerParams.__dataclass_fields__:
        cp = dataclasses.replace(cp, needs_layout_passes=False)

    @functools.partial(
        pl.kernel,
        out_type=jax.ShapeDtypeStruct((OUT_ROWS, D), jnp.float32),
        mesh=mesh,
        compiler_params=cp,
        scratch_types=[
            pltpu.VMEM((CH,), jnp.int32),
            pltpu.VMEM((CH,), jnp.int32),
            pltpu.VMEM((CH,), jnp.int32),
            pltpu.VMEM((CH,), jnp.int32),
            pltpu.VMEM((CH,), jnp.int32),
            pltpu.VMEM((CH, D), jnp.float32),
            pltpu.VMEM((CH, D), jnp.float32),
            pltpu.VMEM((CH, D), jnp.float32),
            pltpu.VMEM_SHARED((NPAD, D), jnp.float32),
            pltpu.VMEM_SHARED((DROWS, D), jnp.float32),
            pltpu.SemaphoreType.DMA,
            pltpu.SemaphoreType.DMA,
            pltpu.SemaphoreType.DMA,
        ],
    )
    def k(x_hbm, src_hbm, dst_hbm, out_hbm,
          sidx0, sidx1, didx0, didx1, dmod, rows0, rows1, hot,
          agg_sh, deg_sh, sem0, sem1, semi):
        cid = lax.axis_index("c")
        sid = lax.axis_index("s")
        wid = sid * NC + cid
        reg = cid * REG

        zero16 = jnp.zeros((16,), jnp.float32)
        one16 = jnp.ones((16,), jnp.float32)
        lane = lax.iota(jnp.int32, 16)

        @pl.loop(0, CH)
        def _(r):
            @pl.loop(0, D // 16)
            def _(c16):
                rows0[r, pl.ds(c16 * 16, 16)] = zero16
                hot[r, pl.ds(c16 * 16, 16)] = zero16

        for t in range(STR // CH):
            pltpu.sync_copy(rows0, agg_sh.at[pl.ds(sid * STR + t * CH, CH)])
        for t in range(DSTR // CH + 1):
            nrow = CH if (t + 1) * CH <= DSTR else DSTR - t * CH
            pltpu.sync_copy(rows0.at[pl.ds(0, nrow)],
                            deg_sh.at[pl.ds(sid * DSTR + t * CH, nrow)])
        plsc.subcore_barrier()

        def idx_base(cnum):
            return jnp.minimum(wid * EPT + cnum * CH, E - CH)

        def start_idx(cnum, sb, db):
            base = idx_base(cnum)
            pltpu.async_copy(src_hbm.at[pl.ds(base, CH)], sb, semi)
            pltpu.async_copy(dst_hbm.at[pl.ds(base, CH)], db, semi)

        def wait_idx(cnum, sb, db):
            base = idx_base(cnum)
            pltpu.make_async_copy(src_hbm.at[pl.ds(base, CH)], sb, semi).wait()
            pltpu.make_async_copy(dst_hbm.at[pl.ds(base, CH)], db, semi).wait()

        def scatter_chunk(rb, db):
            pltpu.sync_copy(rb, agg_sh.at[db], add=True)
            for g in range(CH // 16):
                dv = db[pl.ds(g * 16, 16)]
                dmod[pl.ds(g * 16, 16)] = lax.rem(dv, DM)
                col = (lax.div(dv, DM) * 16).astype(jnp.int32)
                row = lane + g * 16
                plsc.store_scatter(hot, [row, col], one16)
            pltpu.sync_copy(hot, deg_sh.at[dmod], add=True)
            for g in range(CH // 16):
                dv = db[pl.ds(g * 16, 16)]
                col = (lax.div(dv, DM) * 16).astype(jnp.int32)
                row = lane + g * 16
                plsc.store_scatter(hot, [row, col], zero16)

        pltpu.sync_copy(src_hbm.at[pl.ds(idx_base(0), CH)], sidx0)
        pltpu.sync_copy(dst_hbm.at[pl.ds(idx_base(0), CH)], didx0)
        pltpu.async_copy(x_hbm.at[sidx0], rows0, sem0)
        start_idx(1, sidx1, didx1)

        @pl.loop(0, (NCHUNK - 1) // 2)
        def _(t):
            c = 2 * t
            pltpu.make_async_copy(x_hbm.at[sidx0], rows0, sem0).wait()
            wait_idx(c + 1, sidx1, didx1)
            pltpu.async_copy(x_hbm.at[sidx1], rows1, sem1)
            scatter_chunk(rows0, didx0)
            start_idx(c + 2, sidx0, didx0)

            pltpu.make_async_copy(x_hbm.at[sidx1], rows1, sem1).wait()
            wait_idx(c + 2, sidx0, didx0)
            pltpu.async_copy(x_hbm.at[sidx0], rows0, sem0)
            scatter_chunk(rows1, didx1)
            start_idx(c + 3, sidx1, didx1)

        pltpu.make_async_copy(x_hbm.at[sidx0], rows0, sem0).wait()
        wait_idx(NCHUNK, sidx1, didx1)
        scatter_chunk(rows0, didx0)

        plsc.subcore_barrier()

        pltpu.sync_copy(agg_sh.at[pl.ds(sid * STR, STR)],
                        out_hbm.at[pl.ds(reg + sid * STR, STR)])
        pltpu.sync_copy(deg_sh.at[pl.ds(sid * DSTR, DSTR)],
                        out_hbm.at[pl.ds(reg + AOFF + sid * DSTR, DSTR)])

    return k(x, src, dst)


def _tc_dense(x, parts, W, b, gamma, beta):
    BR = 1000
    G = N // BR
    RB = REG // BR
    AB = AOFF // BR

    def body(x_ref, a0_ref, a1_ref, d0_ref, d1_ref, w_ref, b_ref, g_ref,
             be_ref, o_ref):
        i = pl.program_id(0)
        grp = i // 2
        lane_grp = jax.lax.broadcasted_iota(jnp.int32, (BR, D), 1) // 16
        mask = (lane_grp == grp).astype(jnp.float32)
        deg = jnp.sum((d0_ref[...] + d1_ref[...]) * mask, axis=-1,
                      keepdims=True)
        deg = jnp.maximum(deg, 1.0)
        agg = (a0_ref[...] + a1_ref[...]) / deg
        w = w_ref[...]
        h = lax.dot_general(x_ref[...], w[:, :D], (((1,), (1,)), ((), ())),
                            preferred_element_type=jnp.float32)
        h = h + lax.dot_general(agg, w[:, D:], (((1,), (1,)), ((), ())),
                                preferred_element_type=jnp.float32)
        h = jnp.maximum(h + b_ref[...], 0.0)
        mu = jnp.mean(h, axis=-1, keepdims=True)
        hc = h - mu
        var = jnp.mean(hc * hc, axis=-1, keepdims=True)
        out = hc * lax.rsqrt(var + 1e-5)
        o_ref[...] = out * g_ref[...] + be_ref[...]

    return pl.pallas_call(
        body,
        grid=(G,),
        in_specs=[
            pl.BlockSpec((BR, D), lambda i: (i, 0)),
            pl.BlockSpec((BR, D), lambda i: (i, 0)),
            pl.BlockSpec((BR, D), lambda i: (i + RB, 0)),
            pl.BlockSpec((BR, D), lambda i: (AB + i % 2, 0)),
            pl.BlockSpec((BR, D), lambda i: (AB + RB + i % 2, 0)),
            pl.BlockSpec((D, 2 * D), lambda i: (0, 0)),
            pl.BlockSpec((1, D), lambda i: (0, 0)),
            pl.BlockSpec((1, D), lambda i: (0, 0)),
            pl.BlockSpec((1, D), lambda i: (0, 0)),
        ],
        out_specs=pl.BlockSpec((BR, D), lambda i: (i, 0)),
        out_shape=jax.ShapeDtypeStruct((N, D), jnp.float32),
    )(x, parts, parts, parts, parts, W, b.reshape(1, D), gamma.reshape(1, D),
      beta.reshape(1, D))


def kernel(x, edge_index, W, b, gamma, beta):
    ei = edge_index.astype(jnp.int32)
    parts = _sc_aggregate(x, ei[1], ei[0])
    return _tc_dense(x, parts, W, b, gamma, beta)


# speedup vs baseline: 9.1444x; 1.1407x over previous
import dataclasses
import functools

import jax
import jax.numpy as jnp
from jax import lax
from jax.experimental import pallas as pl
from jax.experimental.pallas import tpu as pltpu
from jax.experimental.pallas import tpu_sc as plsc

N = 10000
E = 320000
D = 128
NC = 2
NS = 16
NW = NC * NS
EPT = E // NW
CH = 80
NCHUNK = EPT // CH
STR = 640
NPAD = NS * STR
DM = 2000
DROWS = 2048
DSTR = DROWS // NS
REG = 14000
AOFF = 11000
OUT_ROWS = NC * REG


def _sc_aggregate(x, src, dst):
    mesh = plsc.VectorSubcoreMesh(core_axis_name="c", subcore_axis_name="s")
    cp = pltpu.CompilerParams()
    if "needs_layout_passes" in pltpu.CompilerParams.__dataclass_fields__:
        cp = dataclasses.replace(cp, needs_layout_passes=False)

    @functools.partial(
        pl.kernel,
        out_type=jax.ShapeDtypeStruct((OUT_ROWS, D), jnp.float32),
        mesh=mesh,
        compiler_params=cp,
        scratch_types=[
            pltpu.VMEM((CH,), jnp.int32),
            pltpu.VMEM((CH,), jnp.int32),
            pltpu.VMEM((CH,), jnp.int32),
            pltpu.VMEM((CH,), jnp.int32),
            pltpu.VMEM((CH,), jnp.int32),
            pltpu.VMEM((CH, D), jnp.float32),
            pltpu.VMEM((CH, D), jnp.float32),
            pltpu.VMEM((CH, D), jnp.float32),
            pltpu.VMEM_SHARED((NPAD, D), jnp.float32),
            pltpu.VMEM_SHARED((DROWS, D), jnp.float32),
            pltpu.SemaphoreType.DMA,
            pltpu.SemaphoreType.DMA,
            pltpu.SemaphoreType.DMA,
            pltpu.SemaphoreType.DMA,
        ],
    )
    def k(x_hbm, src_hbm, dst_hbm, out_hbm,
          sidx0, sidx1, didx0, didx1, dmod, rows0, rows1, hot,
          agg_sh, deg_sh, sem0, sem1, semi, sema):
        cid = lax.axis_index("c")
        sid = lax.axis_index("s")
        wid = sid * NC + cid
        reg = cid * REG

        zero16 = jnp.zeros((16,), jnp.float32)
        one16 = jnp.ones((16,), jnp.float32)
        lane = lax.iota(jnp.int32, 16)

        @pl.loop(0, CH)
        def _(r):
            @pl.loop(0, D // 16)
            def _(c16):
                rows0[r, pl.ds(c16 * 16, 16)] = zero16
                hot[r, pl.ds(c16 * 16, 16)] = zero16

        for t in range(STR // CH):
            pltpu.sync_copy(rows0, agg_sh.at[pl.ds(sid * STR + t * CH, CH)])
        for t in range(DSTR // CH + 1):
            nrow = CH if (t + 1) * CH <= DSTR else DSTR - t * CH
            pltpu.sync_copy(rows0.at[pl.ds(0, nrow)],
                            deg_sh.at[pl.ds(sid * DSTR + t * CH, nrow)])
        plsc.subcore_barrier()

        def idx_base(cnum):
            return jnp.minimum(wid * EPT + cnum * CH, E - CH)

        def start_idx(cnum, sb, db):
            base = idx_base(cnum)
            pltpu.async_copy(src_hbm.at[pl.ds(base, CH)], sb, semi)
            pltpu.async_copy(dst_hbm.at[pl.ds(base, CH)], db, semi)

        def wait_idx(cnum, sb, db):
            base = idx_base(cnum)
            pltpu.make_async_copy(src_hbm.at[pl.ds(base, CH)], sb, semi).wait()
            pltpu.make_async_copy(dst_hbm.at[pl.ds(base, CH)], db, semi).wait()

        def scatter_chunk(rb, db):
            pltpu.async_copy(rb, agg_sh.at[db], sema, add=True)
            for g in range(CH // 16):
                dv = db[pl.ds(g * 16, 16)]
                dmod[pl.ds(g * 16, 16)] = lax.rem(dv, DM)
                col = (lax.div(dv, DM) * 16).astype(jnp.int32)
                row = lane + g * 16
                plsc.store_scatter(hot, [row, col], one16)
            pltpu.sync_copy(hot, deg_sh.at[dmod], add=True)
            for g in range(CH // 16):
                dv = db[pl.ds(g * 16, 16)]
                col = (lax.div(dv, DM) * 16).astype(jnp.int32)
                row = lane + g * 16
                plsc.store_scatter(hot, [row, col], zero16)
            pltpu.make_async_copy(rb, agg_sh.at[db], sema).wait()

        pltpu.sync_copy(src_hbm.at[pl.ds(idx_base(0), CH)], sidx0)
        pltpu.sync_copy(dst_hbm.at[pl.ds(idx_base(0), CH)], didx0)
        pltpu.async_copy(x_hbm.at[sidx0], rows0, sem0)
        start_idx(1, sidx1, didx1)

        @pl.loop(0, (NCHUNK - 1) // 2)
        def _(t):
            c = 2 * t
            pltpu.make_async_copy(x_hbm.at[sidx0], rows0, sem0).wait()
            wait_idx(c + 1, sidx1, didx1)
            pltpu.async_copy(x_hbm.at[sidx1], rows1, sem1)
            scatter_chunk(rows0, didx0)
            start_idx(c + 2, sidx0, didx0)

            pltpu.make_async_copy(x_hbm.at[sidx1], rows1, sem1).wait()
            wait_idx(c + 2, sidx0, didx0)
            pltpu.async_copy(x_hbm.at[sidx0], rows0, sem0)
            scatter_chunk(rows1, didx1)
            start_idx(c + 3, sidx1, didx1)

        pltpu.make_async_copy(x_hbm.at[sidx0], rows0, sem0).wait()
        wait_idx(NCHUNK, sidx1, didx1)
        scatter_chunk(rows0, didx0)

        plsc.subcore_barrier()

        pltpu.sync_copy(agg_sh.at[pl.ds(sid * STR, STR)],
                        out_hbm.at[pl.ds(reg + sid * STR, STR)])
        pltpu.sync_copy(deg_sh.at[pl.ds(sid * DSTR, DSTR)],
                        out_hbm.at[pl.ds(reg + AOFF + sid * DSTR, DSTR)])

    return k(x, src, dst)


def _tc_dense(x, parts, W, b, gamma, beta):
    BR = 1000
    G = N // BR
    RB = REG // BR
    AB = AOFF // BR

    def body(x_ref, a0_ref, a1_ref, d0_ref, d1_ref, w_ref, b_ref, g_ref,
             be_ref, o_ref):
        i = pl.program_id(0)
        grp = i // 2
        lane_grp = jax.lax.broadcasted_iota(jnp.int32, (BR, D), 1) // 16
        mask = (lane_grp == grp).astype(jnp.float32)
        deg = jnp.sum((d0_ref[...] + d1_ref[...]) * mask, axis=-1,
                      keepdims=True)
        deg = jnp.maximum(deg, 1.0)
        agg = (a0_ref[...] + a1_ref[...]) / deg
        w = w_ref[...]
        h = lax.dot_general(x_ref[...], w[:, :D], (((1,), (1,)), ((), ())),
                            preferred_element_type=jnp.float32)
        h = h + lax.dot_general(agg, w[:, D:], (((1,), (1,)), ((), ())),
                                preferred_element_type=jnp.float32)
        h = jnp.maximum(h + b_ref[...], 0.0)
        mu = jnp.mean(h, axis=-1, keepdims=True)
        hc = h - mu
        var = jnp.mean(hc * hc, axis=-1, keepdims=True)
        out = hc * lax.rsqrt(var + 1e-5)
        o_ref[...] = out * g_ref[...] + be_ref[...]

    return pl.pallas_call(
        body,
        grid=(G,),
        in_specs=[
            pl.BlockSpec((BR, D), lambda i: (i, 0)),
            pl.BlockSpec((BR, D), lambda i: (i, 0)),
            pl.BlockSpec((BR, D), lambda i: (i + RB, 0)),
            pl.BlockSpec((BR, D), lambda i: (AB + i % 2, 0)),
            pl.BlockSpec((BR, D), lambda i: (AB + RB + i % 2, 0)),
            pl.BlockSpec((D, 2 * D), lambda i: (0, 0)),
            pl.BlockSpec((1, D), lambda i: (0, 0)),
            pl.BlockSpec((1, D), lambda i: (0, 0)),
            pl.BlockSpec((1, D), lambda i: (0, 0)),
        ],
        out_specs=pl.BlockSpec((BR, D), lambda i: (i, 0)),
        out_shape=jax.ShapeDtypeStruct((N, D), jnp.float32),
    )(x, parts, parts, parts, parts, W, b.reshape(1, D), gamma.reshape(1, D),
      beta.reshape(1, D))


def kernel(x, edge_index, W, b, gamma, beta):
    ei = edge_index.astype(jnp.int32)
    parts = _sc_aggregate(x, ei[1], ei[0])
    return _tc_dense(x, parts, W, b, gamma, beta)
